# baseline (speedup 1.0000x reference)
import functools

import jax
import jax.numpy as jnp
from jax import lax
from jax.experimental import pallas as pl
from jax.experimental.pallas import tpu as pltpu
from jax.experimental.pallas import tpu_sc as plsc

D_MODEL = 64
SCALE = 8.0

NC = 2
NS = 16
NW = NC * NS
CHUNK = 128
UNITS = 6400
UPW = UNITS // NW


def _u_to_j_ib(f):
    jb = f >> 8
    ib = (f >> 3) & 31
    s = f & 7
    return jb * 8 + s, ib


@functools.lru_cache(maxsize=None)
def _build():
    mesh = plsc.VectorSubcoreMesh(core_axis_name="c", subcore_axis_name="s")

    @functools.partial(
        pl.kernel,
        mesh=mesh,
        out_type=jax.ShapeDtypeStruct((200, 8, 32, 8, 128), jnp.float32),
        scratch_types=[
            pltpu.VMEM((UPW, CHUNK), jnp.int32),
            pltpu.VMEM((2, CHUNK), jnp.int32),
            pltpu.VMEM((2, CHUNK, 136), jnp.float32),
            pltpu.VMEM((2, 8, 8, 128), jnp.float32),
            pltpu.SemaphoreType.DMA((2,)),
            pltpu.SemaphoreType.DMA((2,)),
        ],
        compiler_params=pltpu.CompilerParams(needs_layout_passes=False),
    )
    def emb(x_hbm, tab_hbm, out_hbm, idx_v, q_v, g_v, t_v, gsem, psem):
        wid = lax.axis_index("s") * NC + lax.axis_index("c")
        pltpu.sync_copy(x_hbm.at[wid], idx_v)

        iota = lax.iota(jnp.int32, 16)

        def gather_start(u, b):
            for m in range(CHUNK // 16):
                sl = pl.ds(m * 16, 16)
                q_v[b, sl] = lax.shift_right_logical(idx_v[u, sl], 1)
            pltpu.async_copy(tab_hbm.at[q_v.at[b]], g_v.at[b, :, pl.ds(0, 128)], gsem.at[b])

        def gather_wait(u, b):
            pltpu.make_async_copy(
                tab_hbm.at[q_v.at[b]], g_v.at[b, :, pl.ds(0, 128)], gsem.at[b]
            ).wait()

        def put_start(u, b):
            j, ib = _u_to_j_ib(wid * UPW + u)
            for cb in range(8):
                pltpu.async_copy(t_v.at[b, cb], out_hbm.at[j, cb, ib], psem.at[b])

        def put_wait(u, b):
            j, ib = _u_to_j_ib(wid * UPW + u)
            for cb in range(8):
                pltpu.make_async_copy(
                    t_v.at[b, cb], out_hbm.at[j, cb, ib], psem.at[b]
                ).wait()

        def transform(u, b):
            rows = [iota + lg * 16 for lg in range(CHUNK // 16)]
            pars = [
                lax.shift_left(idx_v[u, pl.ds(lg * 16, 16)] & 1, 6)
                for lg in range(CHUNK // 16)
            ]

            @plsc.parallel_loop(0, 8, 1)
            def cb_loop(cb):
                cb8 = cb * 8
                for s2 in range(8):
                    for lg in range(CHUNK // 16):
                        g = plsc.load_gather(
                            g_v.at[b], [rows[lg], pars[lg] + (cb8 + s2)]
                        )
                        t_v[b, cb, s2, pl.ds(lg * 16, 16)] = g * SCALE

        for u in range(2):
            gather_start(u, u)
        for u in range(2):
            gather_wait(u, u)
            transform(u, u)
            put_start(u, u)
            gather_start(u + 2, u)

        def steady(g, carry):
            for b in range(2):
                u = 2 * g + b
                gather_wait(u, b)
                put_wait(u - 2, b)
                transform(u, b)
                put_start(u, b)
                gather_start(u + 2, b)
            return carry

        lax.fori_loop(1, UPW // 2 - 1, steady, 0)

        for b in range(2):
            u = UPW - 2 + b
            gather_wait(u, b)
            put_wait(u - 2, b)
            transform(u, b)
            put_start(u, b)
        for b in range(2):
            put_wait(UPW - 2 + b, b)

    return emb


def kernel(x, table):
    xm = (x.astype(jnp.int32)
          .reshape(32, 128, 25, 8)
          .transpose(2, 0, 3, 1)
          .reshape(NW, UPW, CHUNK))
    tr = table.reshape(500000, 128)
    n5 = _build()(xm, tr)
    return n5.transpose(2, 4, 0, 1, 3).reshape(4096, 200, 64)


# speedup vs baseline: 1.0035x; 1.0035x over previous
import functools

import jax
import jax.numpy as jnp
from jax import lax
from jax.experimental import pallas as pl
from jax.experimental.pallas import tpu as pltpu
from jax.experimental.pallas import tpu_sc as plsc

D_MODEL = 64
SCALE = 8.0

NC = 2
NS = 16
NW = NC * NS
CHUNK = 128
UNITS = 6400
UPW = UNITS // NW


def _u_to_j_ib(f):
    jb = f >> 8
    ib = (f >> 3) & 31
    s = f & 7
    return jb * 8 + s, ib


@functools.lru_cache(maxsize=None)
def _build():
    mesh = plsc.VectorSubcoreMesh(core_axis_name="c", subcore_axis_name="s")

    @functools.partial(
        pl.kernel,
        mesh=mesh,
        out_type=jax.ShapeDtypeStruct((200, 8, 32, 8, 128), jnp.float32),
        scratch_types=[
            pltpu.VMEM((UPW, CHUNK), jnp.int32),
            pltpu.VMEM((2, CHUNK), jnp.int32),
            pltpu.VMEM((2, CHUNK, 136), jnp.float32),
            pltpu.VMEM((2, 8, 8, 128), jnp.float32),
            pltpu.SemaphoreType.DMA((2,)),
            pltpu.SemaphoreType.DMA((2,)),
        ],
        compiler_params=pltpu.CompilerParams(needs_layout_passes=False),
    )
    def emb(x_hbm, tab_hbm, out_hbm, idx_v, q_v, g_v, t_v, gsem, psem):
        wid = lax.axis_index("s") * NC + lax.axis_index("c")
        pltpu.sync_copy(x_hbm.at[wid], idx_v)

        iota = lax.iota(jnp.int32, 16)

        def gather_start(u, b):
            for m in range(CHUNK // 16):
                sl = pl.ds(m * 16, 16)
                q_v[b, sl] = lax.shift_right_logical(idx_v[u, sl], 1)
            pltpu.async_copy(tab_hbm.at[q_v.at[b]], g_v.at[b, :, pl.ds(0, 128)], gsem.at[b])

        def gather_wait(u, b):
            pltpu.make_async_copy(
                tab_hbm.at[q_v.at[b]], g_v.at[b, :, pl.ds(0, 128)], gsem.at[b]
            ).wait()

        def put_start(u, b):
            j, ib = _u_to_j_ib(wid * UPW + u)
            pltpu.async_copy(t_v.at[b], out_hbm.at[j, :, ib], psem.at[b])

        def put_wait(u, b):
            j, ib = _u_to_j_ib(wid * UPW + u)
            pltpu.make_async_copy(
                t_v.at[b], out_hbm.at[j, :, ib], psem.at[b]
            ).wait()

        def transform(u, b):
            rows = [iota + lg * 16 for lg in range(CHUNK // 16)]
            pars = [
                lax.shift_left(idx_v[u, pl.ds(lg * 16, 16)] & 1, 6)
                for lg in range(CHUNK // 16)
            ]

            @plsc.parallel_loop(0, 8, 1)
            def cb_loop(cb):
                cb8 = cb * 8
                for s2 in range(8):
                    for lg in range(CHUNK // 16):
                        g = plsc.load_gather(
                            g_v.at[b], [rows[lg], pars[lg] + (cb8 + s2)]
                        )
                        t_v[b, cb, s2, pl.ds(lg * 16, 16)] = g * SCALE

        for u in range(2):
            gather_start(u, u)
        for u in range(2):
            gather_wait(u, u)
            transform(u, u)
            put_start(u, u)
            gather_start(u + 2, u)

        def steady(g, carry):
            for b in range(2):
                u = 2 * g + b
                gather_wait(u, b)
                put_wait(u - 2, b)
                transform(u, b)
                put_start(u, b)
                gather_start(u + 2, b)
            return carry

        lax.fori_loop(1, UPW // 2 - 1, steady, 0)

        for b in range(2):
            u = UPW - 2 + b
            gather_wait(u, b)
            put_wait(u - 2, b)
            transform(u, b)
            put_start(u, b)
        for b in range(2):
            put_wait(UPW - 2 + b, b)

    return emb


def kernel(x, table):
    xm = (x.astype(jnp.int32)
          .reshape(32, 128, 25, 8)
          .transpose(2, 0, 3, 1)
          .reshape(NW, UPW, CHUNK))
    tr = table.reshape(500000, 128)
    n5 = _build()(xm, tr)
    return n5.transpose(2, 4, 0, 1, 3).reshape(4096, 200, 64)


# speedup vs baseline: 1.1286x; 1.1246x over previous
import functools

import jax
import jax.numpy as jnp
from jax import lax
from jax.experimental import pallas as pl
from jax.experimental.pallas import tpu as pltpu
from jax.experimental.pallas import tpu_sc as plsc

D_MODEL = 64
SCALE = 8.0

NC = 2
NS = 16
NW = NC * NS
CHUNK = 128
DEPTH = 4


@functools.lru_cache(maxsize=None)
def _build(nchunks: int):
    assert nchunks % DEPTH == 0 and nchunks // DEPTH >= 3
    mesh = plsc.VectorSubcoreMesh(core_axis_name="c", subcore_axis_name="s")

    @functools.partial(
        pl.kernel,
        mesh=mesh,
        out_type=jax.ShapeDtypeStruct((NW, nchunks, CHUNK, D_MODEL), jnp.float32),
        scratch_types=[
            pltpu.VMEM((nchunks, CHUNK), jnp.int32),
            pltpu.VMEM((DEPTH, CHUNK, D_MODEL), jnp.float32),
            pltpu.VMEM((DEPTH, CHUNK, D_MODEL), jnp.float32),
            pltpu.SemaphoreType.DMA((DEPTH,)),
            pltpu.SemaphoreType.DMA((DEPTH,)),
        ],
        compiler_params=pltpu.CompilerParams(
            use_tc_tiling_on_sc=False, needs_layout_passes=False
        ),
    )
    def emb(x_hbm, table_hbm, out_hbm, idx_v, in_v, out_v, gsem, psem):
        wid = lax.axis_index("s") * NC + lax.axis_index("c")
        pltpu.sync_copy(x_hbm.at[wid], idx_v)

        def gather_start(c, b):
            pltpu.async_copy(table_hbm.at[idx_v.at[c]], in_v.at[b], gsem.at[b])

        def gather_wait(c, b):
            pltpu.make_async_copy(
                table_hbm.at[idx_v.at[c]], in_v.at[b], gsem.at[b]
            ).wait()

        def put_start(c, b):
            pltpu.async_copy(out_v.at[b], out_hbm.at[wid, c], psem.at[b])

        def put_wait(c, b):
            pltpu.make_async_copy(
                out_v.at[b], out_hbm.at[wid, c], psem.at[b]
            ).wait()

        def scale(b):
            @plsc.parallel_loop(0, CHUNK, 1, unroll=4)
            def row(i):
                for j in range(D_MODEL // 16):
                    sl = pl.ds(j * 16, 16)
                    out_v[b, i, sl] = in_v[b, i, sl] * SCALE

        for b in range(DEPTH):
            gather_start(b, b)
        for b in range(DEPTH):
            gather_wait(b, b)
            scale(b)
            put_start(b, b)
            gather_start(b + DEPTH, b)

        def steady(g, carry):
            for b in range(DEPTH):
                c = DEPTH * g + b
                gather_wait(c, b)
                put_wait(c - DEPTH, b)
                scale(b)
                put_start(c, b)
                gather_start(c + DEPTH, b)
            return carry

        lax.fori_loop(1, nchunks // DEPTH - 1, steady, 0)

        tail = nchunks - DEPTH
        for b in range(DEPTH):
            c = tail + b
            gather_wait(c, b)
            put_wait(c - DEPTH, b)
            scale(b)
            put_start(c, b)
        for b in range(DEPTH):
            put_wait(tail + b, b)

    return emb


def kernel(x, table):
    s0, s1 = x.shape
    total = s0 * s1
    assert total % (NW * CHUNK) == 0
    nchunks = total // (NW * CHUNK)
    xr = x.astype(jnp.int32).reshape(NW, nchunks, CHUNK)
    out = _build(nchunks)(xr, table)
    return out.reshape(s0, s1, D_MODEL)
